# Initial kernel scaffold; baseline (speedup 1.0000x reference)
import jax
import jax.numpy as jnp
from jax.experimental import pallas as pl


def kernel(batch, pool_w, W, b):
    raise NotImplementedError("write your pallas kernel here")


# speedup vs baseline: 6.2669x; 6.2669x over previous
import functools
import jax
import jax.numpy as jnp
import numpy as np
from jax.experimental import pallas as pl
from jax.experimental.pallas import tpu as pltpu

_B, _S, _D = 32, 2048, 1024
_V = 100000
_VP = 100352
_LS = 131072
_TEMP = np.float32(0.7)
_TOPP = np.float32(0.5)
_N = _B * _V
_GROWS = 25264


def _zred(x2, nch, rows):
    acc = x2[:, 0, :]
    for c in range(1, nch):
        acc = acc + x2[:, c, :]
    a = acc.reshape(rows, 16, 8)
    acc2 = a[:, 0, :]
    for g in range(1, 16):
        acc2 = acc2 + a[:, g, :]
    t = acc2
    for sh in (4, 2, 1):
        t = t[:, :sh] + t[:, sh:2 * sh]
    return t


def _pool_body(b_ref, w_ref, o_ref):
    row = b_ref[0]
    wmat8 = jnp.broadcast_to(w_ref[0][None, :], (8, _D))
    scores = jnp.dot(wmat8, row.T, preferred_element_type=jnp.float32)[0:1]
    m = jnp.max(scores)
    e = jnp.exp(scores - m)
    z = _zred(e.reshape(1, _S // 128, 128), _S // 128, 1)
    attn = (e / z).astype(jnp.bfloat16)
    rb = row.astype(jnp.bfloat16)
    pooled = jnp.dot(attn, rb, preferred_element_type=jnp.float32)
    o_ref[0] = pooled.astype(jnp.bfloat16)


def _pool(batch, pool_w):
    return pl.pallas_call(
        _pool_body,
        grid=(_B,),
        in_specs=[pl.BlockSpec((1, _S, _D), lambda i: (i, 0, 0)),
                  pl.BlockSpec((1, _D), lambda i: (0, 0))],
        out_specs=pl.BlockSpec((1, 1, _D), lambda i: (i, 0, 0)),
        out_shape=jax.ShapeDtypeStruct((_B, 1, _D), jnp.bfloat16),
    )(batch, pool_w.reshape(1, _D))


def _proj_body(p_ref, w_ref, b_ref, o_ref):
    o_ref[...] = _TEMP * (
        jnp.dot(p_ref[...], w_ref[...], preferred_element_type=jnp.float32)
        + b_ref[...])


def _project(pooled, Wp, bp, block=2048):
    nblk = _VP // block
    return pl.pallas_call(
        _proj_body,
        grid=(nblk,),
        in_specs=[pl.BlockSpec((_B, _D), lambda j: (0, 0)),
                  pl.BlockSpec((_D, block), lambda j: (0, j)),
                  pl.BlockSpec((1, block), lambda j: (0, j))],
        out_specs=pl.BlockSpec((_B, block), lambda j: (0, j)),
        out_shape=jax.ShapeDtypeStruct((_B, _VP), jnp.float32),
    )(pooled, Wp, bp)


def _smax_body(x_ref, o_ref):
    x = x_ref[...]
    m = jnp.max(x, axis=1, keepdims=True)
    e = jnp.exp(x - m)
    z = _zred(e.reshape(_B, _VP // 128, 128), _VP // 128, _B)
    o_ref[...] = e / z


def _softmax_v(logits):
    return pl.pallas_call(
        _smax_body,
        in_specs=[pl.BlockSpec((_B, _VP), lambda: (0, 0))],
        out_specs=pl.BlockSpec((_B, _VP), lambda: (0, 0)),
        out_shape=jax.ShapeDtypeStruct((_B, _VP), jnp.float32),
    )(logits)


def _sort_body(p_ref, k_ref, i_ref):
    kf = -p_ref[0]
    idx = jax.lax.broadcasted_iota(jnp.int32, (_LS // 128, 128), 0) * 128 \
        + jax.lax.broadcasted_iota(jnp.int32, (_LS // 128, 128), 1)
    fi = idx
    kk, ii = kf, idx

    def pshift(x, j):
        if j < 128:
            up = jnp.roll(x, -j, axis=1)
            dn = jnp.roll(x, j, axis=1)
        else:
            up = jnp.roll(x, -(j // 128), axis=0)
            dn = jnp.roll(x, j // 128, axis=0)
        return up, dn

    nbits = 17
    for kpow in range(1, nbits + 1):
        msize = 1 << kpow
        j = msize >> 1
        while j >= 1:
            low = (fi & j) == 0
            ku, kd = pshift(kk, j)
            iu, idn = pshift(ii, j)
            pk = jnp.where(low, ku, kd)
            pi = jnp.where(low, iu, idn)
            a_gt_b = (kk > pk) | ((kk == pk) & (ii > pi))
            dir_up = (fi & msize) == 0
            min_slot = jnp.equal(low, dir_up)
            take_a = jnp.equal(min_slot, ~a_gt_b)
            kk = jnp.where(take_a, kk, pk)
            ii = jnp.where(take_a, ii, pi)
            j >>= 1
    k_ref[0] = kk
    i_ref[0] = ii


def _sort(probs_ext):
    return pl.pallas_call(
        _sort_body,
        grid=(_B,),
        in_specs=[pl.BlockSpec((1, _LS // 128, 128), lambda i: (i, 0, 0))],
        out_specs=[pl.BlockSpec((1, _LS // 128, 128), lambda i: (i, 0, 0)),
                   pl.BlockSpec((1, _LS // 128, 128), lambda i: (i, 0, 0))],
        out_shape=[jax.ShapeDtypeStruct((_B, _LS // 128, 128), jnp.float32),
                   jax.ShapeDtypeStruct((_B, _LS // 128, 128), jnp.int32)],
    )(probs_ext)


def _cumsum_body(k_ref, lp_ref):
    sp = -k_ref[0]
    fi = jax.lax.broadcasted_iota(jnp.int32, (_LS // 128, 128), 0) * 128 \
        + jax.lax.broadcasted_iota(jnp.int32, (_LS // 128, 128), 1)
    valid = fi < _V
    x = jnp.where(valid, sp, 0.0)
    xt = x.T
    rows = [xt[0:1]]
    for l in range(1, 128):
        rows.append(rows[-1] + xt[l:l + 1])
    l0t = jnp.concatenate(rows, axis=0)
    t0 = l0t[127:128]
    t08 = t0.reshape(8, 128).T
    rows1 = [t08[0:1]]
    for l in range(1, 128):
        rows1.append(rows1[-1] + t08[l:l + 1])
    l1t = jnp.concatenate(rows1, axis=0)
    t1 = l1t[127:128]
    c2 = [jnp.zeros((1, 1), jnp.float32)]
    for gidx in range(1, 8):
        c2.append(c2[-1] + t1[:, gidx - 1:gidx])
    c2v = jnp.concatenate(c2, axis=1)
    carry_mid = l1t + c2v
    cm = carry_mid.T.reshape(1, 1024)
    carry0 = jnp.concatenate([jnp.zeros((1, 1), jnp.float32), cm[:, :-1]], axis=1)
    cs_t = l0t + carry0
    cs = cs_t.T
    mask = (cs <= _TOPP) & valid
    lp_ref[0] = jnp.where(mask, jnp.log(sp), -jnp.inf)


def _cumsum_logp(keys):
    return pl.pallas_call(
        _cumsum_body,
        grid=(_B,),
        in_specs=[pl.BlockSpec((1, _LS // 128, 128), lambda i: (i, 0, 0))],
        out_specs=pl.BlockSpec((1, _LS // 128, 128), lambda i: (i, 0, 0)),
        out_shape=jax.ShapeDtypeStruct((_B, _LS // 128, 128), jnp.float32),
    )(keys)


def _pick_body(lp_ref, si_ref, g_ref, o_ref, state):
    b = pl.program_id(0)

    @pl.when(b == 0)
    def _init():
        state[0] = jnp.float32(-jnp.inf)
        state[1] = jnp.float32(0)
        state[2] = jnp.float32(0)

    lp = lp_ref[0]
    si = si_ref[0]
    kcount = jnp.sum((lp > -jnp.inf).astype(jnp.int32))
    off = state[2].astype(jnp.int32)
    base = off // 128
    phi = off % 128
    gseg = g_ref[pl.ds(base, 1040), :]
    x = gseg
    for t in range(7):
        s = 1 << t
        lanes = jax.lax.broadcasted_iota(jnp.int32, (1040, 128), 1)
        a = jnp.roll(x, -s, axis=1)
        bshift = jnp.roll(jnp.roll(x, -1, axis=0), -s, axis=1)
        shifted = jnp.where(lanes < 128 - s, a, bshift)
        cond = ((phi >> t) & 1) == 1
        x = jnp.where(cond, shifted, x)
    score = x[:1024, :] + lp
    fi = jax.lax.broadcasted_iota(jnp.int32, (_LS // 128, 128), 0) * 128 \
        + jax.lax.broadcasted_iota(jnp.int32, (_LS // 128, 128), 1)
    row_best = jnp.max(score)
    at_best = score == row_best
    jstar = jnp.min(jnp.where(at_best, fi, _LS))
    token = jnp.sum(jnp.where(fi == jstar, si, 0))
    better = row_best > state[0]
    state[0] = jnp.where(better, row_best, state[0])
    state[1] = jnp.where(better, token.astype(jnp.float32), state[1])
    state[2] = (off + kcount).astype(jnp.float32)

    @pl.when(b == _B - 1)
    def _fin():
        o_ref[...] = jnp.broadcast_to(state[1].astype(jnp.int32), (1, 1))


def _pick(logp, sidx, g):
    return pl.pallas_call(
        _pick_body,
        grid=(_B,),
        in_specs=[pl.BlockSpec((1, _LS // 128, 128), lambda i: (i, 0, 0)),
                  pl.BlockSpec((1, _LS // 128, 128), lambda i: (i, 0, 0)),
                  pl.BlockSpec((_GROWS, 128), lambda i: (0, 0))],
        out_specs=pl.BlockSpec((1, 1), lambda i: (0, 0)),
        out_shape=jax.ShapeDtypeStruct((1, 1), jnp.int32),
        scratch_shapes=[pltpu.SMEM((4,), jnp.float32)],
    )(logp, sidx, g)


def kernel(batch, pool_w, W, b):
    pooled = _pool(batch, pool_w)[:, 0].astype(jnp.float32)
    Wp = jnp.pad(W, ((0, 0), (0, _VP - _V)))
    bp = jnp.pad(b, (0, _VP - _V), constant_values=-jnp.inf).reshape(1, _VP)
    logits = _project(pooled, Wp, bp)
    probs = _softmax_v(logits)
    probs_ext = jnp.pad(probs, ((0, 0), (0, _LS - _VP))).reshape(_B, _LS // 128, 128)
    keys, sidx = _sort(probs_ext)
    logp = _cumsum_logp(keys)
    skey = jax.random.key(42)
    u = jax.random.uniform(skey, (1, _N), jnp.float32,
                           minval=jnp.finfo(jnp.float32).tiny, maxval=1.)[0]
    g = -jnp.log(-jnp.log(u))
    g = jnp.pad(g, (0, _GROWS * 128 - _N)).reshape(_GROWS, 128)
    token = _pick(logp, sidx, g)
    return token[0, 0]


# speedup vs baseline: 6.2688x; 1.0003x over previous
import jax
import jax.numpy as jnp
import numpy as np
from jax.experimental import pallas as pl
from jax.experimental.pallas import tpu as pltpu

_B, _S, _D = 32, 2048, 1024
_V = 100000
_VP = 100352
_LS = 131072
_TEMP = np.float32(0.7)
_TOPP = np.float32(0.5)
_N = _B * _V
_GROWS = 25264


def _zred(x2, nch, rows):
    acc = x2[:, 0, :]
    for c in range(1, nch):
        acc = acc + x2[:, c, :]
    a = acc.reshape(rows, 16, 8)
    acc2 = a[:, 0, :]
    for g in range(1, 16):
        acc2 = acc2 + a[:, g, :]
    t = acc2
    for sh in (4, 2, 1):
        t = t[:, :sh] + t[:, sh:2 * sh]
    return t


def _pool_body(b_ref, w_ref, o_ref):
    row = b_ref[0]
    wmat8 = jnp.broadcast_to(w_ref[0][None, :], (8, _D))
    scores = jnp.dot(wmat8, row.T, preferred_element_type=jnp.float32)[0:1]
    m = jnp.max(scores)
    e = jnp.exp(scores - m)
    z = _zred(e.reshape(1, _S // 128, 128), _S // 128, 1)
    attn = (e / z).astype(jnp.bfloat16)
    rb = row.astype(jnp.bfloat16)
    pooled = jnp.dot(attn, rb, preferred_element_type=jnp.float32)
    o_ref[0] = pooled.astype(jnp.bfloat16)


def _pool(batch, pool_w):
    return pl.pallas_call(
        _pool_body,
        grid=(_B,),
        in_specs=[pl.BlockSpec((1, _S, _D), lambda i: (i, 0, 0)),
                  pl.BlockSpec((1, _D), lambda i: (0, 0))],
        out_specs=pl.BlockSpec((1, 1, _D), lambda i: (i, 0, 0)),
        out_shape=jax.ShapeDtypeStruct((_B, 1, _D), jnp.bfloat16),
    )(batch, pool_w.reshape(1, _D))


def _proj_body(p_ref, w_ref, b_ref, o_ref):
    o_ref[...] = _TEMP * (
        jnp.dot(p_ref[...], w_ref[...], preferred_element_type=jnp.float32)
        + b_ref[...])


def _project(pooled, Wp, bp, block=2048):
    nblk = _VP // block
    return pl.pallas_call(
        _proj_body,
        grid=(nblk,),
        in_specs=[pl.BlockSpec((_B, _D), lambda j: (0, 0)),
                  pl.BlockSpec((_D, block), lambda j: (0, j)),
                  pl.BlockSpec((1, block), lambda j: (0, j))],
        out_specs=pl.BlockSpec((_B, block), lambda j: (0, j)),
        out_shape=jax.ShapeDtypeStruct((_B, _VP), jnp.float32),
    )(pooled, Wp, bp)


def _smax_body(x_ref, o_ref):
    x = x_ref[...]
    m = jnp.max(x, axis=1, keepdims=True)
    e = jnp.exp(x - m)
    z = _zred(e.reshape(_B, _VP // 128, 128), _VP // 128, _B)
    o_ref[...] = e / z


def _softmax_v(logits):
    return pl.pallas_call(
        _smax_body,
        in_specs=[pl.BlockSpec((_B, _VP), lambda: (0, 0))],
        out_specs=pl.BlockSpec((_B, _VP), lambda: (0, 0)),
        out_shape=jax.ShapeDtypeStruct((_B, _VP), jnp.float32),
    )(logits)


def _sort_body(p_ref, k_ref, i_ref):
    kf = -p_ref[0]
    idx = jax.lax.broadcasted_iota(jnp.int32, (_LS // 128, 128), 0) * 128 \
        + jax.lax.broadcasted_iota(jnp.int32, (_LS // 128, 128), 1)
    fi = idx
    kk, ii = kf, idx

    def pshift(x, j):
        if j < 128:
            up = jnp.roll(x, -j, axis=1)
            dn = jnp.roll(x, j, axis=1)
        else:
            up = jnp.roll(x, -(j // 128), axis=0)
            dn = jnp.roll(x, j // 128, axis=0)
        return up, dn

    nbits = 17
    for kpow in range(1, nbits + 1):
        msize = 1 << kpow
        j = msize >> 1
        while j >= 1:
            low = (fi & j) == 0
            ku, kd = pshift(kk, j)
            iu, idn = pshift(ii, j)
            pk = jnp.where(low, ku, kd)
            pi = jnp.where(low, iu, idn)
            a_gt_b = (kk > pk) | ((kk == pk) & (ii > pi))
            dir_up = (fi & msize) == 0
            min_slot = jnp.equal(low, dir_up)
            take_a = jnp.equal(min_slot, ~a_gt_b)
            kk = jnp.where(take_a, kk, pk)
            ii = jnp.where(take_a, ii, pi)
            j >>= 1
    k_ref[0] = kk
    i_ref[0] = ii


def _sort(probs_ext):
    return pl.pallas_call(
        _sort_body,
        grid=(_B,),
        in_specs=[pl.BlockSpec((1, _LS // 128, 128), lambda i: (i, 0, 0))],
        out_specs=[pl.BlockSpec((1, _LS // 128, 128), lambda i: (i, 0, 0)),
                   pl.BlockSpec((1, _LS // 128, 128), lambda i: (i, 0, 0))],
        out_shape=[jax.ShapeDtypeStruct((_B, _LS // 128, 128), jnp.float32),
                   jax.ShapeDtypeStruct((_B, _LS // 128, 128), jnp.int32)],
    )(probs_ext)


def _cumsum_body(k_ref, lp_ref):
    sp = -k_ref[0]
    fi = jax.lax.broadcasted_iota(jnp.int32, (_LS // 128, 128), 0) * 128 \
        + jax.lax.broadcasted_iota(jnp.int32, (_LS // 128, 128), 1)
    valid = fi < _V
    x = jnp.where(valid, sp, 0.0)
    xt = x.T
    rows = [xt[0:1]]
    for l in range(1, 128):
        rows.append(rows[-1] + xt[l:l + 1])
    l0t = jnp.concatenate(rows, axis=0)
    t0 = l0t[127:128]
    t08 = t0.reshape(8, 128).T
    rows1 = [t08[0:1]]
    for l in range(1, 128):
        rows1.append(rows1[-1] + t08[l:l + 1])
    l1t = jnp.concatenate(rows1, axis=0)
    t1 = l1t[127:128]
    c2 = [jnp.zeros((1, 1), jnp.float32)]
    for gidx in range(1, 8):
        c2.append(c2[-1] + t1[:, gidx - 1:gidx])
    c2v = jnp.concatenate(c2, axis=1)
    carry_mid = l1t + c2v
    cm = carry_mid.T.reshape(1, 1024)
    carry0 = jnp.concatenate([jnp.zeros((1, 1), jnp.float32), cm[:, :-1]], axis=1)
    cs_t = l0t + carry0
    cs = cs_t.T
    mask = (cs <= _TOPP) & valid
    lp_ref[0] = jnp.where(mask, jnp.log(sp), -jnp.inf)


def _cumsum_logp(keys):
    return pl.pallas_call(
        _cumsum_body,
        grid=(_B,),
        in_specs=[pl.BlockSpec((1, _LS // 128, 128), lambda i: (i, 0, 0))],
        out_specs=pl.BlockSpec((1, _LS // 128, 128), lambda i: (i, 0, 0)),
        out_shape=jax.ShapeDtypeStruct((_B, _LS // 128, 128), jnp.float32),
    )(keys)


def _pick_body(lp_ref, si_ref, g_ref, o_ref, state):
    b = pl.program_id(0)

    @pl.when(b == 0)
    def _init():
        state[0] = jnp.float32(-jnp.inf)
        state[1] = jnp.float32(0)
        state[2] = jnp.float32(0)

    lp = lp_ref[0]
    si = si_ref[0]
    kcount = jnp.sum((lp > -jnp.inf).astype(jnp.int32))
    off = state[2].astype(jnp.int32)
    base = off // 128
    phi = off % 128
    gseg = g_ref[pl.ds(base, 1040), :]
    x = gseg
    for t in range(7):
        s = 1 << t
        lanes = jax.lax.broadcasted_iota(jnp.int32, (1040, 128), 1)
        a = jnp.roll(x, -s, axis=1)
        bshift = jnp.roll(jnp.roll(x, -1, axis=0), -s, axis=1)
        shifted = jnp.where(lanes < 128 - s, a, bshift)
        cond = ((phi >> t) & 1) == 1
        x = jnp.where(cond, shifted, x)
    score = x[:1024, :] + lp
    fi = jax.lax.broadcasted_iota(jnp.int32, (_LS // 128, 128), 0) * 128 \
        + jax.lax.broadcasted_iota(jnp.int32, (_LS // 128, 128), 1)
    row_best = jnp.max(score)
    at_best = score == row_best
    jstar = jnp.min(jnp.where(at_best, fi, _LS))
    token = jnp.sum(jnp.where(fi == jstar, si, 0))
    better = row_best > state[0]
    state[0] = jnp.where(better, row_best, state[0])
    state[1] = jnp.where(better, token.astype(jnp.float32), state[1])
    state[2] = (off + kcount).astype(jnp.float32)

    @pl.when(b == _B - 1)
    def _fin():
        o_ref[...] = jnp.broadcast_to(state[1].astype(jnp.int32), (1, 1))


def _pick(logp, sidx, g):
    return pl.pallas_call(
        _pick_body,
        grid=(_B,),
        in_specs=[pl.BlockSpec((1, _LS // 128, 128), lambda i: (i, 0, 0)),
                  pl.BlockSpec((1, _LS // 128, 128), lambda i: (i, 0, 0)),
                  pl.BlockSpec((_GROWS, 128), lambda i: (0, 0))],
        out_specs=pl.BlockSpec((1, 1), lambda i: (0, 0)),
        out_shape=jax.ShapeDtypeStruct((1, 1), jnp.int32),
        scratch_shapes=[pltpu.SMEM((4,), jnp.float32)],
    )(logp, sidx, g)


def kernel(batch, pool_w, W, b):
    pooled = _pool(batch, pool_w)[:, 0].astype(jnp.float32)
    Wp = jnp.pad(W, ((0, 0), (0, _VP - _V)))
    bp = jnp.pad(b, (0, _VP - _V), constant_values=-jnp.inf).reshape(1, _VP)
    logits = _project(pooled, Wp, bp)
    probs = _softmax_v(logits)
    probs_ext = jnp.pad(probs, ((0, 0), (0, _LS - _VP))).reshape(_B, _LS // 128, 128)
    keys, sidx = _sort(probs_ext)
    logp = _cumsum_logp(keys)
    skey = jax.random.key(42)
    u = jax.random.uniform(skey, (1, _N), jnp.float32,
                           minval=jnp.finfo(jnp.float32).tiny, maxval=1.)[0]
    g = -jnp.log(-jnp.log(u))
    g = jnp.pad(g, (0, _GROWS * 128 - _N)).reshape(_GROWS, 128)
    token = _pick(logp, sidx, g)
    return token[0, 0]
